# baseline (speedup 1.0000x reference)
import numpy as np
import jax
import jax.numpy as jnp
from jax import lax
from jax.experimental import pallas as pl
from jax.experimental.pallas import tpu as pltpu
from jax.experimental.pallas import tpu_sc as plsc

_N = 50000
_M = 50000
_E = 20
_TOT = _N + _M
_B = 16384

_BL = 2048
_GRID = (_TOT + _BL - 1) // _BL
_CW = 128

_BP = 2048
_PGRID = _B // _BP

_NW = 32
_PPW = _B // _NW
_CHUNK = 128

_LO = np.float32(np.nextafter(np.float32(-1.0), np.float32(0.0)))
_SPAN = np.float32(np.float32(1.0) - _LO)
_SQRT2 = np.float32(np.sqrt(np.float32(2.0)))


def _threefry_bits(k0, k1, cnt):
    ks2 = k0 ^ k1 ^ jnp.uint32(0x1BD11BDA)
    x0 = jnp.zeros_like(cnt) + k0
    x1 = cnt + k1
    rot1 = (13, 15, 26, 6)
    rot2 = (17, 29, 16, 24)

    def rounds(x0, x1, rots):
        for r in rots:
            x0 = x0 + x1
            x1 = lax.shift_left(x1, jnp.uint32(r)) | lax.shift_right_logical(
                x1, jnp.uint32(32 - r))
            x1 = x0 ^ x1
        return x0, x1

    x0, x1 = rounds(x0, x1, rot1)
    x0 = x0 + k1
    x1 = x1 + ks2 + jnp.uint32(1)
    x0, x1 = rounds(x0, x1, rot2)
    x0 = x0 + ks2
    x1 = x1 + k0 + jnp.uint32(2)
    x0, x1 = rounds(x0, x1, rot1)
    x0 = x0 + k0
    x1 = x1 + k1 + jnp.uint32(3)
    x0, x1 = rounds(x0, x1, rot2)
    x0 = x0 + k1
    x1 = x1 + ks2 + jnp.uint32(4)
    x0, x1 = rounds(x0, x1, rot1)
    x0 = x0 + ks2
    x1 = x1 + k0 + jnp.uint32(5)
    return x0 ^ x1


def _bits_to_normal(bits):
    fl = lax.bitcast_convert_type(
        lax.shift_right_logical(bits, jnp.uint32(9)) | jnp.uint32(0x3F800000),
        jnp.float32) - np.float32(1.0)
    u = jnp.maximum(_LO, fl * _SPAN + _LO)
    return _SQRT2 * lax.erf_inv(u)


def _kl_body(scal_ref, up_ref, ip_ref, bias_ref, ent_ref,
             klb_ref, kle_ref, klg_ref, std_ref):
    pid = pl.program_id(0)
    sp = jax.nn.softplus
    alpha = scal_ref[0]
    gbm = scal_ref[1]
    gbs = scal_ref[2]
    prec_g = scal_ref[3]
    prec_ub = scal_ref[4]
    prec_ib = scal_ref[5]

    gb_scale = sp(gbs)
    prior_g = sp(prec_g)
    klg_ref[...] = jnp.full((1, 1), jnp.log(prior_g / gb_scale)
                            + (gb_scale * gb_scale + gbm * gbm) / (2.0 * prior_g * prior_g)
                            - 0.5, jnp.float32)
    std_ref[...] = jnp.full((1, 1), jnp.sqrt(1.0 / sp(alpha)), jnp.float32)

    lane = pid * _BL + lax.broadcasted_iota(jnp.int32, (1, _BL), 1)
    is_user = lane < _N

    bT = bias_ref[...].T
    bl = bT[0:1, :]
    bs = sp(bT[1:2, :])
    pbu = sp(prec_ub)
    pbi = sp(prec_ib)
    logpb = jnp.where(is_user, jnp.log(pbu), jnp.log(pbi))
    wb = jnp.where(is_user, 1.0 / (2.0 * pbu * pbu), 1.0 / (2.0 * pbi * pbi))
    klb_ref[...] = (logpb - jnp.log(bs) + (bs * bs + bl * bl) * wb - 0.5).reshape(_BL)

    eT = ent_ref[...].T
    loc = eT[0:_E, :]
    esc = sp(eT[_E:, :])
    pu = sp(up_ref[...])
    pi_ = sp(ip_ref[...])
    w = jnp.where(is_user, 1.0 / (2.0 * pu * pu), 1.0 / (2.0 * pi_ * pi_))
    logp = jnp.where(is_user, jnp.sum(jnp.log(pu)), jnp.sum(jnp.log(pi_)))
    f = (esc * esc + loc * loc) * w - jnp.log(esc)
    kle_ref[...] = (jnp.sum(f, axis=0, keepdims=True) + (logp - 0.5 * _E)).reshape(_BL)


def _sc_body(comb_hbm, iu_hbm, ii_hbm, uout_hbm, iout_hbm, iu_v, ii_v, rows, sem):
    c = lax.axis_index("c")
    s = lax.axis_index("s")
    wid = s * 2 + c
    base = wid * _PPW
    pltpu.sync_copy(iu_hbm.at[pl.ds(base, _PPW)], iu_v)
    pltpu.sync_copy(ii_hbm.at[pl.ds(base, _PPW)], ii_v)

    for idx_v, out_hbm in ((iu_v, uout_hbm), (ii_v, iout_hbm)):
        copies = []
        for j in range(_PPW // _CHUNK):
            sl = pl.ds(j * _CHUNK, _CHUNK)
            copies.append(pltpu.async_copy(comb_hbm.at[idx_v.at[sl]], rows.at[sl], sem))
        for cp in copies:
            cp.wait()
        pltpu.sync_copy(rows, out_hbm.at[pl.ds(base, _PPW), :])


def _pred_body(scal_ref, keys_ref, u_ref, i_ref, iu_ref, ii_ref, out_ref):
    sp = jax.nn.softplus
    gbm = scal_ref[1]
    gbs = scal_ref[2]
    eps_g = scal_ref[6]
    global_bias = gbm + sp(gbs) * eps_g

    k2a = keys_ref[0]
    k2b = keys_ref[1]
    k3a = keys_ref[2]
    k3b = keys_ref[3]

    def side(rows_ref, idx_ref):
        rT = rows_ref[...].T
        loc = rT[0:_E, :]
        scp = rT[_E:2 * _E, :]
        bloc = rT[2 * _E:2 * _E + 1, :]
        bscp = rT[2 * _E + 1:2 * _E + 2, :]
        idx = lax.convert_element_type(idx_ref[...], jnp.uint32)
        cnt_e = idx * jnp.uint32(_E) + lax.broadcasted_iota(jnp.uint32, (_E, _BP), 0)
        eps_e = _bits_to_normal(_threefry_bits(k3a, k3b, cnt_e))
        eps_b = _bits_to_normal(_threefry_bits(k2a, k2b, idx))
        ae = loc + sp(scp) * eps_e
        ab = bloc + sp(bscp) * eps_b
        return ae, ab

    ae_u, ab_u = side(u_ref, iu_ref)
    ae_i, ab_i = side(i_ref, ii_ref)
    out_ref[...] = (jnp.sum(ae_u * ae_i, axis=0, keepdims=True)
                    + ab_u + ab_i + global_bias).reshape(_BP)


def _gather_rows(comb0, iu, ii):
    mesh = plsc.VectorSubcoreMesh(core_axis_name="c", subcore_axis_name="s")
    return pl.kernel(
        _sc_body,
        out_type=[jax.ShapeDtypeStruct((_B, _CW), jnp.float32),
                  jax.ShapeDtypeStruct((_B, _CW), jnp.float32)],
        mesh=mesh,
        scratch_types=[
            pltpu.VMEM((_PPW,), jnp.int32),
            pltpu.VMEM((_PPW,), jnp.int32),
            pltpu.VMEM((_PPW, _CW), jnp.float32),
            pltpu.SemaphoreType.DMA,
        ],
    )(comb0, iu, ii)


def kernel(x, bias_table, entity_table, alpha, global_bias_mean, global_bias_scale,
           prec_global_bias_prior, prec_user_bias_prior, prec_item_bias_prior,
           prec_user_entity_prior, prec_item_entity_prior):
    ek1, ek2, ek3 = jax.random.split(jax.random.key(42), 3)
    eps_g = jax.random.normal(ek1, (1, 1), dtype=jnp.float32)
    keys = jnp.concatenate([jax.random.key_data(ek2),
                            jax.random.key_data(ek3)]).astype(jnp.uint32)

    scal = jnp.concatenate([
        alpha.reshape(1).astype(jnp.float32),
        global_bias_mean.reshape(1).astype(jnp.float32),
        global_bias_scale.reshape(1).astype(jnp.float32),
        prec_global_bias_prior.reshape(1).astype(jnp.float32),
        prec_user_bias_prior.reshape(1).astype(jnp.float32),
        prec_item_bias_prior.reshape(1).astype(jnp.float32),
        eps_g.reshape(1),
        jnp.zeros((1,), jnp.float32),
    ])

    ftab = entity_table.astype(jnp.float32)
    fbias = bias_table.astype(jnp.float32)
    up_t = prec_user_entity_prior.astype(jnp.float32).reshape(_E, 1)
    ip_t = prec_item_entity_prior.astype(jnp.float32).reshape(_E, 1)

    klb, kle, klg, std = pl.pallas_call(
        _kl_body,
        grid=(_GRID,),
        in_specs=[
            pl.BlockSpec(memory_space=pltpu.SMEM),
            pl.BlockSpec((_E, 1), lambda i: (0, 0)),
            pl.BlockSpec((_E, 1), lambda i: (0, 0)),
            pl.BlockSpec((_BL, 2), lambda i: (i, 0)),
            pl.BlockSpec((_BL, 2 * _E), lambda i: (i, 0)),
        ],
        out_specs=[
            pl.BlockSpec((_BL,), lambda i: (i,)),
            pl.BlockSpec((_BL,), lambda i: (i,)),
            pl.BlockSpec((1, 1), lambda i: (0, 0)),
            pl.BlockSpec((1, 1), lambda i: (0, 0)),
        ],
        out_shape=[
            jax.ShapeDtypeStruct((_TOT,), jnp.float32),
            jax.ShapeDtypeStruct((_TOT,), jnp.float32),
            jax.ShapeDtypeStruct((1, 1), jnp.float32),
            jax.ShapeDtypeStruct((1, 1), jnp.float32),
        ],
    )(scal, up_t, ip_t, fbias, ftab)

    comb0 = jnp.concatenate(
        [ftab, fbias, jnp.zeros((_TOT, _CW - 2 * _E - 2), jnp.float32)], axis=1)
    iu = x[:, 0].astype(jnp.int32)
    ii = x[:, 1].astype(jnp.int32)
    u_rows, i_rows = _gather_rows(comb0, iu, ii)

    pred = pl.pallas_call(
        _pred_body,
        grid=(_PGRID,),
        in_specs=[
            pl.BlockSpec(memory_space=pltpu.SMEM),
            pl.BlockSpec(memory_space=pltpu.SMEM),
            pl.BlockSpec((_BP, _CW), lambda i: (i, 0)),
            pl.BlockSpec((_BP, _CW), lambda i: (i, 0)),
            pl.BlockSpec((1, _BP), lambda i: (0, i)),
            pl.BlockSpec((1, _BP), lambda i: (0, i)),
        ],
        out_specs=pl.BlockSpec((_BP,), lambda i: (i,)),
        out_shape=jax.ShapeDtypeStruct((_B,), jnp.float32),
    )(scal, keys, u_rows, i_rows, iu.reshape(1, _B), ii.reshape(1, _B))

    return (pred,
            std.reshape(1),
            klg.reshape(1),
            klb,
            kle)


# speedup vs baseline: 1.4300x; 1.4300x over previous
import numpy as np
import jax
import jax.numpy as jnp
from jax import lax
from jax.experimental import pallas as pl
from jax.experimental.pallas import tpu as pltpu
from jax.experimental.pallas import tpu_sc as plsc

_N = 50000
_M = 50000
_E = 20
_TOT = _N + _M
_B = 16384

_BL = 2048
_GRID = (_TOT + _BL - 1) // _BL
_CW = 128

_BP = 2048
_PGRID = _B // _BP

_NW = 32
_PPW = _B // _NW
_CHUNK = 128

_LO = np.float32(np.nextafter(np.float32(-1.0), np.float32(0.0)))
_SPAN = np.float32(np.float32(1.0) - _LO)
_SQRT2 = np.float32(np.sqrt(np.float32(2.0)))


def _threefry_bits(k0, k1, cnt):
    ks2 = k0 ^ k1 ^ jnp.uint32(0x1BD11BDA)
    x0 = jnp.zeros_like(cnt) + k0
    x1 = cnt + k1
    rot1 = (13, 15, 26, 6)
    rot2 = (17, 29, 16, 24)

    def rounds(x0, x1, rots):
        for r in rots:
            x0 = x0 + x1
            x1 = lax.shift_left(x1, jnp.uint32(r)) | lax.shift_right_logical(
                x1, jnp.uint32(32 - r))
            x1 = x0 ^ x1
        return x0, x1

    x0, x1 = rounds(x0, x1, rot1)
    x0 = x0 + k1
    x1 = x1 + ks2 + jnp.uint32(1)
    x0, x1 = rounds(x0, x1, rot2)
    x0 = x0 + ks2
    x1 = x1 + k0 + jnp.uint32(2)
    x0, x1 = rounds(x0, x1, rot1)
    x0 = x0 + k0
    x1 = x1 + k1 + jnp.uint32(3)
    x0, x1 = rounds(x0, x1, rot2)
    x0 = x0 + k1
    x1 = x1 + ks2 + jnp.uint32(4)
    x0, x1 = rounds(x0, x1, rot1)
    x0 = x0 + ks2
    x1 = x1 + k0 + jnp.uint32(5)
    return x0 ^ x1


def _bits_to_normal(bits):
    fl = lax.bitcast_convert_type(
        lax.shift_right_logical(bits, jnp.uint32(9)) | jnp.uint32(0x3F800000),
        jnp.float32) - np.float32(1.0)
    u = jnp.maximum(_LO, fl * _SPAN + _LO)
    return _SQRT2 * lax.erf_inv(u)


def _comb_body(biasT_ref, entT_ref, comb_ref):
    entR = entT_ref[...].T
    biasR = biasT_ref[...].T
    comb_ref[...] = jnp.concatenate(
        [entR, biasR, jnp.zeros((_BL, _CW - 2 * _E - 2), jnp.float32)], axis=1)


def _kl_body(scal_ref, up_ref, ip_ref, biasT_ref, entT_ref,
             klb_ref, kle_ref, klg_ref, std_ref):
    pid = pl.program_id(0)
    sp = jax.nn.softplus
    alpha = scal_ref[0]
    gbm = scal_ref[1]
    gbs = scal_ref[2]
    prec_g = scal_ref[3]
    prec_ub = scal_ref[4]
    prec_ib = scal_ref[5]

    gb_scale = sp(gbs)
    prior_g = sp(prec_g)
    klg_ref[...] = jnp.full((1, 1), jnp.log(prior_g / gb_scale)
                            + (gb_scale * gb_scale + gbm * gbm) / (2.0 * prior_g * prior_g)
                            - 0.5, jnp.float32)
    std_ref[...] = jnp.full((1, 1), jnp.sqrt(1.0 / sp(alpha)), jnp.float32)

    lane = pid * _BL + lax.broadcasted_iota(jnp.int32, (1, _BL), 1)
    is_user = lane < _N

    bl = biasT_ref[0:1, :]
    bs = sp(biasT_ref[1:2, :])
    pbu = sp(prec_ub)
    pbi = sp(prec_ib)
    logpb = jnp.where(is_user, jnp.log(pbu), jnp.log(pbi))
    wb = jnp.where(is_user, 1.0 / (2.0 * pbu * pbu), 1.0 / (2.0 * pbi * pbi))
    klb_ref[...] = (logpb - jnp.log(bs) + (bs * bs + bl * bl) * wb - 0.5).reshape(_BL)

    loc = entT_ref[0:_E, :]
    esc = sp(entT_ref[_E:, :])
    pu = sp(up_ref[...])
    pi_ = sp(ip_ref[...])
    w = jnp.where(is_user, 1.0 / (2.0 * pu * pu), 1.0 / (2.0 * pi_ * pi_))
    logp = jnp.where(is_user, jnp.sum(jnp.log(pu)), jnp.sum(jnp.log(pi_)))
    f = (esc * esc + loc * loc) * w - jnp.log(esc)
    kle_ref[...] = (jnp.sum(f, axis=0, keepdims=True) + (logp - 0.5 * _E)).reshape(_BL)


def _sc_body(comb_hbm, iu_hbm, ii_hbm, uout_hbm, iout_hbm, iu_v, ii_v, rows, sem):
    c = lax.axis_index("c")
    s = lax.axis_index("s")
    wid = s * 2 + c
    base = wid * _PPW
    pltpu.sync_copy(iu_hbm.at[pl.ds(base, _PPW)], iu_v)
    pltpu.sync_copy(ii_hbm.at[pl.ds(base, _PPW)], ii_v)

    for idx_v, out_hbm in ((iu_v, uout_hbm), (ii_v, iout_hbm)):
        copies = []
        for j in range(_PPW // _CHUNK):
            sl = pl.ds(j * _CHUNK, _CHUNK)
            copies.append(pltpu.async_copy(comb_hbm.at[idx_v.at[sl]], rows.at[sl], sem))
        for cp in copies:
            cp.wait()
        pltpu.sync_copy(rows, out_hbm.at[pl.ds(base, _PPW), :])


def _pred_body(scal_ref, keys_ref, u_ref, i_ref, iu_ref, ii_ref, out_ref):
    sp = jax.nn.softplus
    gbm = scal_ref[1]
    gbs = scal_ref[2]
    eps_g = scal_ref[6]
    global_bias = gbm + sp(gbs) * eps_g

    k2a = keys_ref[0]
    k2b = keys_ref[1]
    k3a = keys_ref[2]
    k3b = keys_ref[3]

    def side(rows_ref, idx_ref):
        rT = rows_ref[...].T
        loc = rT[0:_E, :]
        scp = rT[_E:2 * _E, :]
        bloc = rT[2 * _E:2 * _E + 1, :]
        bscp = rT[2 * _E + 1:2 * _E + 2, :]
        idx = lax.convert_element_type(idx_ref[...], jnp.uint32)
        cnt_e = idx * jnp.uint32(_E) + lax.broadcasted_iota(jnp.uint32, (_E, _BP), 0)
        eps_e = _bits_to_normal(_threefry_bits(k3a, k3b, cnt_e))
        eps_b = _bits_to_normal(_threefry_bits(k2a, k2b, idx))
        ae = loc + sp(scp) * eps_e
        ab = bloc + sp(bscp) * eps_b
        return ae, ab

    ae_u, ab_u = side(u_ref, iu_ref)
    ae_i, ab_i = side(i_ref, ii_ref)
    out_ref[...] = (jnp.sum(ae_u * ae_i, axis=0, keepdims=True)
                    + ab_u + ab_i + global_bias).reshape(_BP)


def _gather_rows(comb0, iu, ii):
    mesh = plsc.VectorSubcoreMesh(core_axis_name="c", subcore_axis_name="s")
    return pl.kernel(
        _sc_body,
        out_type=[jax.ShapeDtypeStruct((_B, _CW), jnp.float32),
                  jax.ShapeDtypeStruct((_B, _CW), jnp.float32)],
        mesh=mesh,
        scratch_types=[
            pltpu.VMEM((_PPW,), jnp.int32),
            pltpu.VMEM((_PPW,), jnp.int32),
            pltpu.VMEM((_PPW, _CW), jnp.float32),
            pltpu.SemaphoreType.DMA,
        ],
    )(comb0, iu, ii)


def kernel(x, bias_table, entity_table, alpha, global_bias_mean, global_bias_scale,
           prec_global_bias_prior, prec_user_bias_prior, prec_item_bias_prior,
           prec_user_entity_prior, prec_item_entity_prior):
    ek1, ek2, ek3 = jax.random.split(jax.random.key(42), 3)
    eps_g = jax.random.normal(ek1, (1, 1), dtype=jnp.float32)
    keys = jnp.concatenate([jax.random.key_data(ek2),
                            jax.random.key_data(ek3)]).astype(jnp.uint32)

    scal = jnp.concatenate([
        alpha.reshape(1).astype(jnp.float32),
        global_bias_mean.reshape(1).astype(jnp.float32),
        global_bias_scale.reshape(1).astype(jnp.float32),
        prec_global_bias_prior.reshape(1).astype(jnp.float32),
        prec_user_bias_prior.reshape(1).astype(jnp.float32),
        prec_item_bias_prior.reshape(1).astype(jnp.float32),
        eps_g.reshape(1),
        jnp.zeros((1,), jnp.float32),
    ])

    biasT = bias_table.astype(jnp.float32).T
    entT = entity_table.astype(jnp.float32).T
    up_t = prec_user_entity_prior.astype(jnp.float32).reshape(_E, 1)
    ip_t = prec_item_entity_prior.astype(jnp.float32).reshape(_E, 1)

    comb0 = pl.pallas_call(
        _comb_body,
        grid=(_GRID,),
        in_specs=[
            pl.BlockSpec((2, _BL), lambda i: (0, i)),
            pl.BlockSpec((2 * _E, _BL), lambda i: (0, i)),
        ],
        out_specs=pl.BlockSpec((_BL, _CW), lambda i: (i, 0)),
        out_shape=jax.ShapeDtypeStruct((_TOT, _CW), jnp.float32),
    )(biasT, entT)

    iu = x[:, 0].astype(jnp.int32)
    ii = x[:, 1].astype(jnp.int32)
    u_rows, i_rows = _gather_rows(comb0, iu, ii)

    klb, kle, klg, std = pl.pallas_call(
        _kl_body,
        grid=(_GRID,),
        in_specs=[
            pl.BlockSpec(memory_space=pltpu.SMEM),
            pl.BlockSpec((_E, 1), lambda i: (0, 0)),
            pl.BlockSpec((_E, 1), lambda i: (0, 0)),
            pl.BlockSpec((2, _BL), lambda i: (0, i)),
            pl.BlockSpec((2 * _E, _BL), lambda i: (0, i)),
        ],
        out_specs=[
            pl.BlockSpec((_BL,), lambda i: (i,)),
            pl.BlockSpec((_BL,), lambda i: (i,)),
            pl.BlockSpec((1, 1), lambda i: (0, 0)),
            pl.BlockSpec((1, 1), lambda i: (0, 0)),
        ],
        out_shape=[
            jax.ShapeDtypeStruct((_TOT,), jnp.float32),
            jax.ShapeDtypeStruct((_TOT,), jnp.float32),
            jax.ShapeDtypeStruct((1, 1), jnp.float32),
            jax.ShapeDtypeStruct((1, 1), jnp.float32),
        ],
    )(scal, up_t, ip_t, biasT, entT)

    pred = pl.pallas_call(
        _pred_body,
        grid=(_PGRID,),
        in_specs=[
            pl.BlockSpec(memory_space=pltpu.SMEM),
            pl.BlockSpec(memory_space=pltpu.SMEM),
            pl.BlockSpec((_BP, _CW), lambda i: (i, 0)),
            pl.BlockSpec((_BP, _CW), lambda i: (i, 0)),
            pl.BlockSpec((1, _BP), lambda i: (0, i)),
            pl.BlockSpec((1, _BP), lambda i: (0, i)),
        ],
        out_specs=pl.BlockSpec((_BP,), lambda i: (i,)),
        out_shape=jax.ShapeDtypeStruct((_B,), jnp.float32),
    )(scal, keys, u_rows, i_rows, iu.reshape(1, _B), ii.reshape(1, _B))

    return (pred,
            std.reshape(1),
            klg.reshape(1),
            klb,
            kle)


# speedup vs baseline: 1.6683x; 1.1667x over previous
import numpy as np
import jax
import jax.numpy as jnp
from jax import lax
from jax.experimental import pallas as pl
from jax.experimental.pallas import tpu as pltpu
from jax.experimental.pallas import tpu_sc as plsc

_N = 50000
_M = 50000
_E = 20
_TOT = _N + _M
_B = 16384

_BL = 4096
_GRID = (_TOT + _BL - 1) // _BL
_CW = 128

_BP = 4096
_PGRID = _B // _BP

_NW = 32
_PPW = _B // _NW
_CHUNK = 128

_LO = np.float32(np.nextafter(np.float32(-1.0), np.float32(0.0)))
_SPAN = np.float32(np.float32(1.0) - _LO)
_SQRT2 = np.float32(np.sqrt(np.float32(2.0)))


def _threefry_bits(k0, k1, cnt):
    ks2 = k0 ^ k1 ^ jnp.uint32(0x1BD11BDA)
    x0 = jnp.zeros_like(cnt) + k0
    x1 = cnt + k1
    rot1 = (13, 15, 26, 6)
    rot2 = (17, 29, 16, 24)

    def rounds(x0, x1, rots):
        for r in rots:
            x0 = x0 + x1
            x1 = lax.shift_left(x1, jnp.uint32(r)) | lax.shift_right_logical(
                x1, jnp.uint32(32 - r))
            x1 = x0 ^ x1
        return x0, x1

    x0, x1 = rounds(x0, x1, rot1)
    x0 = x0 + k1
    x1 = x1 + ks2 + jnp.uint32(1)
    x0, x1 = rounds(x0, x1, rot2)
    x0 = x0 + ks2
    x1 = x1 + k0 + jnp.uint32(2)
    x0, x1 = rounds(x0, x1, rot1)
    x0 = x0 + k0
    x1 = x1 + k1 + jnp.uint32(3)
    x0, x1 = rounds(x0, x1, rot2)
    x0 = x0 + k1
    x1 = x1 + ks2 + jnp.uint32(4)
    x0, x1 = rounds(x0, x1, rot1)
    x0 = x0 + ks2
    x1 = x1 + k0 + jnp.uint32(5)
    return x0 ^ x1


def _bits_to_normal(bits):
    fl = lax.bitcast_convert_type(
        lax.shift_right_logical(bits, jnp.uint32(9)) | jnp.uint32(0x3F800000),
        jnp.float32) - np.float32(1.0)
    u = jnp.maximum(_LO, fl * _SPAN + _LO)
    return _SQRT2 * lax.erf_inv(u)


def _comb_body(biasT_ref, entT_ref, comb_ref):
    entR = entT_ref[...].T
    biasR = biasT_ref[...].T
    comb_ref[...] = jnp.concatenate(
        [entR, biasR, jnp.zeros((_BL, _CW - 2 * _E - 2), jnp.float32)], axis=1)


def _kl_body(scal_ref, up_ref, ip_ref, biasT_ref, entT_ref,
             klb_ref, kle_ref, klg_ref, std_ref):
    pid = pl.program_id(0)
    sp = jax.nn.softplus
    alpha = scal_ref[0]
    gbm = scal_ref[1]
    gbs = scal_ref[2]
    prec_g = scal_ref[3]
    prec_ub = scal_ref[4]
    prec_ib = scal_ref[5]

    gb_scale = sp(gbs)
    prior_g = sp(prec_g)
    klg_ref[...] = jnp.full((1, 1), jnp.log(prior_g / gb_scale)
                            + (gb_scale * gb_scale + gbm * gbm) / (2.0 * prior_g * prior_g)
                            - 0.5, jnp.float32)
    std_ref[...] = jnp.full((1, 1), jnp.sqrt(1.0 / sp(alpha)), jnp.float32)

    lane = pid * _BL + lax.broadcasted_iota(jnp.int32, (1, _BL), 1)
    is_user = lane < _N

    bl = biasT_ref[0:1, :]
    bs = sp(biasT_ref[1:2, :])
    pbu = sp(prec_ub)
    pbi = sp(prec_ib)
    logpb = jnp.where(is_user, jnp.log(pbu), jnp.log(pbi))
    wb = jnp.where(is_user, 1.0 / (2.0 * pbu * pbu), 1.0 / (2.0 * pbi * pbi))
    klb_ref[...] = (logpb - jnp.log(bs) + (bs * bs + bl * bl) * wb - 0.5).reshape(_BL)

    loc = entT_ref[0:_E, :]
    esc = sp(entT_ref[_E:, :])
    pu = sp(up_ref[...])
    pi_ = sp(ip_ref[...])
    w = jnp.where(is_user, 1.0 / (2.0 * pu * pu), 1.0 / (2.0 * pi_ * pi_))
    logp = jnp.where(is_user, jnp.sum(jnp.log(pu)), jnp.sum(jnp.log(pi_)))
    f = (esc * esc + loc * loc) * w - jnp.log(esc)
    kle_ref[...] = (jnp.sum(f, axis=0, keepdims=True) + (logp - 0.5 * _E)).reshape(_BL)


def _sc_body(comb_hbm, iu_hbm, ii_hbm, uout_hbm, iout_hbm, iu_v, ii_v, rows, sem):
    c = lax.axis_index("c")
    s = lax.axis_index("s")
    wid = s * 2 + c
    base = wid * _PPW
    pltpu.sync_copy(iu_hbm.at[pl.ds(base, _PPW)], iu_v)
    pltpu.sync_copy(ii_hbm.at[pl.ds(base, _PPW)], ii_v)

    for idx_v, out_hbm in ((iu_v, uout_hbm), (ii_v, iout_hbm)):
        copies = []
        for j in range(_PPW // _CHUNK):
            sl = pl.ds(j * _CHUNK, _CHUNK)
            copies.append(pltpu.async_copy(comb_hbm.at[idx_v.at[sl]], rows.at[sl], sem))
        for cp in copies:
            cp.wait()
        pltpu.sync_copy(rows, out_hbm.at[pl.ds(base, _PPW), :])


def _pred_body(scal_ref, keys_ref, u_ref, i_ref, iu_ref, ii_ref, out_ref):
    sp = jax.nn.softplus
    gbm = scal_ref[1]
    gbs = scal_ref[2]
    eps_g = scal_ref[6]
    global_bias = gbm + sp(gbs) * eps_g

    k2a = keys_ref[0]
    k2b = keys_ref[1]
    k3a = keys_ref[2]
    k3b = keys_ref[3]

    def side(rows_ref, idx_ref):
        rT = rows_ref[...].T
        loc = rT[0:_E, :]
        scp = rT[_E:2 * _E, :]
        bloc = rT[2 * _E:2 * _E + 1, :]
        bscp = rT[2 * _E + 1:2 * _E + 2, :]
        idx = lax.convert_element_type(idx_ref[...], jnp.uint32)
        cnt_e = idx * jnp.uint32(_E) + lax.broadcasted_iota(jnp.uint32, (_E, _BP), 0)
        eps_e = _bits_to_normal(_threefry_bits(k3a, k3b, cnt_e))
        eps_b = _bits_to_normal(_threefry_bits(k2a, k2b, idx))
        ae = loc + sp(scp) * eps_e
        ab = bloc + sp(bscp) * eps_b
        return ae, ab

    ae_u, ab_u = side(u_ref, iu_ref)
    ae_i, ab_i = side(i_ref, ii_ref)
    out_ref[...] = (jnp.sum(ae_u * ae_i, axis=0, keepdims=True)
                    + ab_u + ab_i + global_bias).reshape(_BP)


def _gather_rows(comb0, iu, ii):
    mesh = plsc.VectorSubcoreMesh(core_axis_name="c", subcore_axis_name="s")
    return pl.kernel(
        _sc_body,
        out_type=[jax.ShapeDtypeStruct((_B, _CW), jnp.float32),
                  jax.ShapeDtypeStruct((_B, _CW), jnp.float32)],
        mesh=mesh,
        scratch_types=[
            pltpu.VMEM((_PPW,), jnp.int32),
            pltpu.VMEM((_PPW,), jnp.int32),
            pltpu.VMEM((_PPW, _CW), jnp.float32),
            pltpu.SemaphoreType.DMA,
        ],
    )(comb0, iu, ii)


def kernel(x, bias_table, entity_table, alpha, global_bias_mean, global_bias_scale,
           prec_global_bias_prior, prec_user_bias_prior, prec_item_bias_prior,
           prec_user_entity_prior, prec_item_entity_prior):
    ek1, ek2, ek3 = jax.random.split(jax.random.key(42), 3)
    eps_g = jax.random.normal(ek1, (1, 1), dtype=jnp.float32)
    keys = jnp.concatenate([jax.random.key_data(ek2),
                            jax.random.key_data(ek3)]).astype(jnp.uint32)

    scal = jnp.concatenate([
        alpha.reshape(1).astype(jnp.float32),
        global_bias_mean.reshape(1).astype(jnp.float32),
        global_bias_scale.reshape(1).astype(jnp.float32),
        prec_global_bias_prior.reshape(1).astype(jnp.float32),
        prec_user_bias_prior.reshape(1).astype(jnp.float32),
        prec_item_bias_prior.reshape(1).astype(jnp.float32),
        eps_g.reshape(1),
        jnp.zeros((1,), jnp.float32),
    ])

    biasT = bias_table.astype(jnp.float32).T
    entT = entity_table.astype(jnp.float32).T
    up_t = prec_user_entity_prior.astype(jnp.float32).reshape(_E, 1)
    ip_t = prec_item_entity_prior.astype(jnp.float32).reshape(_E, 1)

    comb0 = pl.pallas_call(
        _comb_body,
        grid=(_GRID,),
        in_specs=[
            pl.BlockSpec((2, _BL), lambda i: (0, i)),
            pl.BlockSpec((2 * _E, _BL), lambda i: (0, i)),
        ],
        out_specs=pl.BlockSpec((_BL, _CW), lambda i: (i, 0)),
        out_shape=jax.ShapeDtypeStruct((_TOT, _CW), jnp.float32),
    )(biasT, entT)

    iu = x[:, 0].astype(jnp.int32)
    ii = x[:, 1].astype(jnp.int32)
    u_rows, i_rows = _gather_rows(comb0, iu, ii)

    klb, kle, klg, std = pl.pallas_call(
        _kl_body,
        grid=(_GRID,),
        in_specs=[
            pl.BlockSpec(memory_space=pltpu.SMEM),
            pl.BlockSpec((_E, 1), lambda i: (0, 0)),
            pl.BlockSpec((_E, 1), lambda i: (0, 0)),
            pl.BlockSpec((2, _BL), lambda i: (0, i)),
            pl.BlockSpec((2 * _E, _BL), lambda i: (0, i)),
        ],
        out_specs=[
            pl.BlockSpec((_BL,), lambda i: (i,)),
            pl.BlockSpec((_BL,), lambda i: (i,)),
            pl.BlockSpec((1, 1), lambda i: (0, 0)),
            pl.BlockSpec((1, 1), lambda i: (0, 0)),
        ],
        out_shape=[
            jax.ShapeDtypeStruct((_TOT,), jnp.float32),
            jax.ShapeDtypeStruct((_TOT,), jnp.float32),
            jax.ShapeDtypeStruct((1, 1), jnp.float32),
            jax.ShapeDtypeStruct((1, 1), jnp.float32),
        ],
    )(scal, up_t, ip_t, biasT, entT)

    pred = pl.pallas_call(
        _pred_body,
        grid=(_PGRID,),
        in_specs=[
            pl.BlockSpec(memory_space=pltpu.SMEM),
            pl.BlockSpec(memory_space=pltpu.SMEM),
            pl.BlockSpec((_BP, _CW), lambda i: (i, 0)),
            pl.BlockSpec((_BP, _CW), lambda i: (i, 0)),
            pl.BlockSpec((1, _BP), lambda i: (0, i)),
            pl.BlockSpec((1, _BP), lambda i: (0, i)),
        ],
        out_specs=pl.BlockSpec((_BP,), lambda i: (i,)),
        out_shape=jax.ShapeDtypeStruct((_B,), jnp.float32),
    )(scal, keys, u_rows, i_rows, iu.reshape(1, _B), ii.reshape(1, _B))

    return (pred,
            std.reshape(1),
            klg.reshape(1),
            klb,
            kle)


# speedup vs baseline: 1.6879x; 1.0117x over previous
import numpy as np
import jax
import jax.numpy as jnp
from jax import lax
from jax.experimental import pallas as pl
from jax.experimental.pallas import tpu as pltpu
from jax.experimental.pallas import tpu_sc as plsc

_N = 50000
_M = 50000
_E = 20
_TOT = _N + _M
_B = 16384

_BL = 8192
_GRID = (_TOT + _BL - 1) // _BL
_CW = 128

_BP = 8192
_PGRID = _B // _BP

_NW = 32
_PPW = _B // _NW
_CHUNK = 128

_LO = np.float32(np.nextafter(np.float32(-1.0), np.float32(0.0)))
_SPAN = np.float32(np.float32(1.0) - _LO)
_SQRT2 = np.float32(np.sqrt(np.float32(2.0)))


def _threefry_bits(k0, k1, cnt):
    ks2 = k0 ^ k1 ^ jnp.uint32(0x1BD11BDA)
    x0 = jnp.zeros_like(cnt) + k0
    x1 = cnt + k1
    rot1 = (13, 15, 26, 6)
    rot2 = (17, 29, 16, 24)

    def rounds(x0, x1, rots):
        for r in rots:
            x0 = x0 + x1
            x1 = lax.shift_left(x1, jnp.uint32(r)) | lax.shift_right_logical(
                x1, jnp.uint32(32 - r))
            x1 = x0 ^ x1
        return x0, x1

    x0, x1 = rounds(x0, x1, rot1)
    x0 = x0 + k1
    x1 = x1 + ks2 + jnp.uint32(1)
    x0, x1 = rounds(x0, x1, rot2)
    x0 = x0 + ks2
    x1 = x1 + k0 + jnp.uint32(2)
    x0, x1 = rounds(x0, x1, rot1)
    x0 = x0 + k0
    x1 = x1 + k1 + jnp.uint32(3)
    x0, x1 = rounds(x0, x1, rot2)
    x0 = x0 + k1
    x1 = x1 + ks2 + jnp.uint32(4)
    x0, x1 = rounds(x0, x1, rot1)
    x0 = x0 + ks2
    x1 = x1 + k0 + jnp.uint32(5)
    return x0 ^ x1


def _bits_to_normal(bits):
    fl = lax.bitcast_convert_type(
        lax.shift_right_logical(bits, jnp.uint32(9)) | jnp.uint32(0x3F800000),
        jnp.float32) - np.float32(1.0)
    u = jnp.maximum(_LO, fl * _SPAN + _LO)
    return _SQRT2 * lax.erf_inv(u)


def _comb_body(biasT_ref, entT_ref, comb_ref):
    entR = entT_ref[...].T
    biasR = biasT_ref[...].T
    comb_ref[...] = jnp.concatenate(
        [entR, biasR, jnp.zeros((_BL, _CW - 2 * _E - 2), jnp.float32)], axis=1)


def _kl_body(scal_ref, up_ref, ip_ref, biasT_ref, entT_ref,
             klb_ref, kle_ref, klg_ref, std_ref):
    pid = pl.program_id(0)
    sp = jax.nn.softplus
    alpha = scal_ref[0]
    gbm = scal_ref[1]
    gbs = scal_ref[2]
    prec_g = scal_ref[3]
    prec_ub = scal_ref[4]
    prec_ib = scal_ref[5]

    gb_scale = sp(gbs)
    prior_g = sp(prec_g)
    klg_ref[...] = jnp.full((1, 1), jnp.log(prior_g / gb_scale)
                            + (gb_scale * gb_scale + gbm * gbm) / (2.0 * prior_g * prior_g)
                            - 0.5, jnp.float32)
    std_ref[...] = jnp.full((1, 1), jnp.sqrt(1.0 / sp(alpha)), jnp.float32)

    lane = pid * _BL + lax.broadcasted_iota(jnp.int32, (1, _BL), 1)
    is_user = lane < _N

    bl = biasT_ref[0:1, :]
    bs = sp(biasT_ref[1:2, :])
    pbu = sp(prec_ub)
    pbi = sp(prec_ib)
    logpb = jnp.where(is_user, jnp.log(pbu), jnp.log(pbi))
    wb = jnp.where(is_user, 1.0 / (2.0 * pbu * pbu), 1.0 / (2.0 * pbi * pbi))
    klb_ref[...] = (logpb - jnp.log(bs) + (bs * bs + bl * bl) * wb - 0.5).reshape(_BL)

    loc = entT_ref[0:_E, :]
    esc = sp(entT_ref[_E:, :])
    pu = sp(up_ref[...])
    pi_ = sp(ip_ref[...])
    w = jnp.where(is_user, 1.0 / (2.0 * pu * pu), 1.0 / (2.0 * pi_ * pi_))
    logp = jnp.where(is_user, jnp.sum(jnp.log(pu)), jnp.sum(jnp.log(pi_)))
    f = (esc * esc + loc * loc) * w - jnp.log(esc)
    kle_ref[...] = (jnp.sum(f, axis=0, keepdims=True) + (logp - 0.5 * _E)).reshape(_BL)


def _sc_body(comb_hbm, iu_hbm, ii_hbm, uout_hbm, iout_hbm, iu_v, ii_v, rows, sem):
    c = lax.axis_index("c")
    s = lax.axis_index("s")
    wid = s * 2 + c
    base = wid * _PPW
    pltpu.sync_copy(iu_hbm.at[pl.ds(base, _PPW)], iu_v)
    pltpu.sync_copy(ii_hbm.at[pl.ds(base, _PPW)], ii_v)

    for idx_v, out_hbm in ((iu_v, uout_hbm), (ii_v, iout_hbm)):
        copies = []
        for j in range(_PPW // _CHUNK):
            sl = pl.ds(j * _CHUNK, _CHUNK)
            copies.append(pltpu.async_copy(comb_hbm.at[idx_v.at[sl]], rows.at[sl], sem))
        for cp in copies:
            cp.wait()
        pltpu.sync_copy(rows, out_hbm.at[pl.ds(base, _PPW), :])


def _pred_body(scal_ref, keys_ref, u_ref, i_ref, iu_ref, ii_ref, out_ref):
    sp = jax.nn.softplus
    gbm = scal_ref[1]
    gbs = scal_ref[2]
    eps_g = scal_ref[6]
    global_bias = gbm + sp(gbs) * eps_g

    k2a = keys_ref[0]
    k2b = keys_ref[1]
    k3a = keys_ref[2]
    k3b = keys_ref[3]

    def side(rows_ref, idx_ref):
        rT = rows_ref[...].T
        loc = rT[0:_E, :]
        scp = rT[_E:2 * _E, :]
        bloc = rT[2 * _E:2 * _E + 1, :]
        bscp = rT[2 * _E + 1:2 * _E + 2, :]
        idx = lax.convert_element_type(idx_ref[...], jnp.uint32)
        cnt_e = idx * jnp.uint32(_E) + lax.broadcasted_iota(jnp.uint32, (_E, _BP), 0)
        eps_e = _bits_to_normal(_threefry_bits(k3a, k3b, cnt_e))
        eps_b = _bits_to_normal(_threefry_bits(k2a, k2b, idx))
        ae = loc + sp(scp) * eps_e
        ab = bloc + sp(bscp) * eps_b
        return ae, ab

    ae_u, ab_u = side(u_ref, iu_ref)
    ae_i, ab_i = side(i_ref, ii_ref)
    out_ref[...] = (jnp.sum(ae_u * ae_i, axis=0, keepdims=True)
                    + ab_u + ab_i + global_bias).reshape(_BP)


def _gather_rows(comb0, iu, ii):
    mesh = plsc.VectorSubcoreMesh(core_axis_name="c", subcore_axis_name="s")
    return pl.kernel(
        _sc_body,
        out_type=[jax.ShapeDtypeStruct((_B, _CW), jnp.float32),
                  jax.ShapeDtypeStruct((_B, _CW), jnp.float32)],
        mesh=mesh,
        scratch_types=[
            pltpu.VMEM((_PPW,), jnp.int32),
            pltpu.VMEM((_PPW,), jnp.int32),
            pltpu.VMEM((_PPW, _CW), jnp.float32),
            pltpu.SemaphoreType.DMA,
        ],
    )(comb0, iu, ii)


def kernel(x, bias_table, entity_table, alpha, global_bias_mean, global_bias_scale,
           prec_global_bias_prior, prec_user_bias_prior, prec_item_bias_prior,
           prec_user_entity_prior, prec_item_entity_prior):
    ek1, ek2, ek3 = jax.random.split(jax.random.key(42), 3)
    eps_g = jax.random.normal(ek1, (1, 1), dtype=jnp.float32)
    keys = jnp.concatenate([jax.random.key_data(ek2),
                            jax.random.key_data(ek3)]).astype(jnp.uint32)

    scal = jnp.concatenate([
        alpha.reshape(1).astype(jnp.float32),
        global_bias_mean.reshape(1).astype(jnp.float32),
        global_bias_scale.reshape(1).astype(jnp.float32),
        prec_global_bias_prior.reshape(1).astype(jnp.float32),
        prec_user_bias_prior.reshape(1).astype(jnp.float32),
        prec_item_bias_prior.reshape(1).astype(jnp.float32),
        eps_g.reshape(1),
        jnp.zeros((1,), jnp.float32),
    ])

    biasT = bias_table.astype(jnp.float32).T
    entT = entity_table.astype(jnp.float32).T
    up_t = prec_user_entity_prior.astype(jnp.float32).reshape(_E, 1)
    ip_t = prec_item_entity_prior.astype(jnp.float32).reshape(_E, 1)

    comb0 = pl.pallas_call(
        _comb_body,
        grid=(_GRID,),
        in_specs=[
            pl.BlockSpec((2, _BL), lambda i: (0, i)),
            pl.BlockSpec((2 * _E, _BL), lambda i: (0, i)),
        ],
        out_specs=pl.BlockSpec((_BL, _CW), lambda i: (i, 0)),
        out_shape=jax.ShapeDtypeStruct((_TOT, _CW), jnp.float32),
    )(biasT, entT)

    iu = x[:, 0].astype(jnp.int32)
    ii = x[:, 1].astype(jnp.int32)
    u_rows, i_rows = _gather_rows(comb0, iu, ii)

    klb, kle, klg, std = pl.pallas_call(
        _kl_body,
        grid=(_GRID,),
        in_specs=[
            pl.BlockSpec(memory_space=pltpu.SMEM),
            pl.BlockSpec((_E, 1), lambda i: (0, 0)),
            pl.BlockSpec((_E, 1), lambda i: (0, 0)),
            pl.BlockSpec((2, _BL), lambda i: (0, i)),
            pl.BlockSpec((2 * _E, _BL), lambda i: (0, i)),
        ],
        out_specs=[
            pl.BlockSpec((_BL,), lambda i: (i,)),
            pl.BlockSpec((_BL,), lambda i: (i,)),
            pl.BlockSpec((1, 1), lambda i: (0, 0)),
            pl.BlockSpec((1, 1), lambda i: (0, 0)),
        ],
        out_shape=[
            jax.ShapeDtypeStruct((_TOT,), jnp.float32),
            jax.ShapeDtypeStruct((_TOT,), jnp.float32),
            jax.ShapeDtypeStruct((1, 1), jnp.float32),
            jax.ShapeDtypeStruct((1, 1), jnp.float32),
        ],
    )(scal, up_t, ip_t, biasT, entT)

    pred = pl.pallas_call(
        _pred_body,
        grid=(_PGRID,),
        in_specs=[
            pl.BlockSpec(memory_space=pltpu.SMEM),
            pl.BlockSpec(memory_space=pltpu.SMEM),
            pl.BlockSpec((_BP, _CW), lambda i: (i, 0)),
            pl.BlockSpec((_BP, _CW), lambda i: (i, 0)),
            pl.BlockSpec((1, _BP), lambda i: (0, i)),
            pl.BlockSpec((1, _BP), lambda i: (0, i)),
        ],
        out_specs=pl.BlockSpec((_BP,), lambda i: (i,)),
        out_shape=jax.ShapeDtypeStruct((_B,), jnp.float32),
    )(scal, keys, u_rows, i_rows, iu.reshape(1, _B), ii.reshape(1, _B))

    return (pred,
            std.reshape(1),
            klg.reshape(1),
            klb,
            kle)
